# baseline (speedup 1.0000x reference)
import functools

import jax
import jax.numpy as jnp
from jax import lax
from jax.experimental import pallas as pl
from jax.experimental.pallas import tpu as pltpu
from jax.experimental.pallas import tpu_sc as plsc

_N_FIELDS = 26
_VOCAB = 100000
_D = 32
_B = 4096


def kernel(x, tables):
    info = plsc.get_sparse_core_info()
    nw = info.num_cores * info.num_subcores
    b_per_w = _B // nw

    mesh = plsc.VectorSubcoreMesh(core_axis_name="c", subcore_axis_name="s")

    @functools.partial(
        pl.kernel,
        mesh=mesh,
        compiler_params=pltpu.CompilerParams(use_tc_tiling_on_sc=False),
        out_type=jax.ShapeDtypeStruct((_B, _N_FIELDS, _D), jnp.float32),
        scratch_types=[
            pltpu.VMEM((_N_FIELDS, b_per_w), jnp.int32),
            pltpu.VMEM((_N_FIELDS, b_per_w, _D), jnp.float32),
            pltpu.SemaphoreType.DMA,
            pltpu.SemaphoreType.DMA,
        ],
    )
    def gather_kernel(x_hbm, tab_hbm, out_hbm, idx_v, rows_v, gsem, wsem):
        wid = lax.axis_index("s") * info.num_cores + lax.axis_index("c")
        base = wid * b_per_w
        pltpu.sync_copy(x_hbm.at[:, pl.ds(base, b_per_w)], idx_v)
        gathers = []
        for f in range(_N_FIELDS):
            tab_f = tab_hbm.at[pl.ds(f * _VOCAB, _VOCAB)]
            gathers.append(
                pltpu.async_copy(tab_f.at[idx_v.at[f]], rows_v.at[f], gsem)
            )
        writes = []
        for f in range(_N_FIELDS):
            gathers[f].wait()
            writes.append(
                pltpu.async_copy(
                    rows_v.at[f], out_hbm.at[pl.ds(base, b_per_w), f], wsem
                )
            )
        for w in writes:
            w.wait()

    return gather_kernel(x, tables.reshape(_N_FIELDS * _VOCAB, _D))


# speedup vs baseline: 3.9364x; 3.9364x over previous
import functools

import jax
import jax.numpy as jnp
from jax import lax
from jax.experimental import pallas as pl
from jax.experimental.pallas import tpu as pltpu
from jax.experimental.pallas import tpu_sc as plsc

_N_FIELDS = 26
_VOCAB = 100000
_D = 32
_B = 4096
_WIN = 2048
_NFULL = _VOCAB // _WIN
_TAIL = _VOCAB - _NFULL * _WIN
_NWIN = _NFULL + 1
_SHIFT = 11
_CHUNKS = _B // 16
_PAIR_CAP = _B + _NWIN * 15 + 9
_PAIR_CHUNKS = _PAIR_CAP // 16
_N_ITEMS = _N_FIELDS * 4
_CNT_BASE = 1


def kernel(x, tables):
    info = plsc.get_sparse_core_info()
    nc = info.num_cores
    nw = nc * info.num_subcores

    mesh = plsc.VectorSubcoreMesh(core_axis_name="c", subcore_axis_name="s")

    @functools.partial(
        pl.kernel,
        mesh=mesh,
        out_type=jax.ShapeDtypeStruct((_N_FIELDS, 4, 8, _B), jnp.float32),
        compiler_params=pltpu.CompilerParams(needs_layout_passes=False),
        scratch_types=[
            pltpu.VMEM((2, 8, _WIN), jnp.float32),
            pltpu.VMEM((8, _TAIL), jnp.float32),
            pltpu.VMEM((8, _B + 8), jnp.float32),
            pltpu.VMEM((_B,), jnp.int32),
            pltpu.VMEM((_PAIR_CAP,), jnp.int32),
            pltpu.VMEM((_PAIR_CAP,), jnp.int32),
            pltpu.VMEM((64,), jnp.int32),
            pltpu.VMEM((64,), jnp.int32),
            pltpu.VMEM((64,), jnp.int32),
            pltpu.SemaphoreType.DMA,
            pltpu.SemaphoreType.DMA,
        ],
    )
    def gather_kernel(x_hbm, tab_hbm, out_hbm, win_v, tail_v, out_v, xrow_v,
                      pv_v, pb_v, hcnt_v, pbase_v, hrun_v, s_win, s_out):
        w = lax.axis_index("s") * nc + lax.axis_index("c")
        iota = lax.iota(jnp.int32, 16)
        zeros16 = jnp.zeros((16,), jnp.int32)

        def extract(vec64_ref, j):
            c = pl.multiple_of((j >> 4) * 16, 8)
            v16 = vec64_ref[pl.ds(c, 16)]
            return jnp.max(jnp.where(iota == (j & 15), v16, 0))

        def fire_window(item, wi, k):
            f = item >> 2
            ds = item & 3
            buf = (wi + k) & 1
            off = pl.multiple_of(wi * _WIN, 128)

            @pl.when(wi < _NFULL)
            def _():
                pltpu.async_copy(
                    tab_hbm.at[f, ds, :, pl.ds(off, _WIN)],
                    win_v.at[buf], s_win)

            @pl.when(wi == _NFULL)
            def _():
                pltpu.async_copy(
                    tab_hbm.at[f, ds, :, pl.ds(_NFULL * _WIN, _TAIL)],
                    tail_v, s_win)

        def drain_window(full):
            if full:
                pltpu.make_async_copy(
                    tab_hbm.at[0, 0, :, pl.ds(0, _WIN)], win_v.at[0],
                    s_win).wait()
            else:
                pltpu.make_async_copy(
                    tab_hbm.at[0, 0, :, pl.ds(_NFULL * _WIN, _TAIL)],
                    tail_v, s_win).wait()

        def gather_window(wi, k):
            gather_from(win_v.at[(wi + k) & 1], wi)

        def gather_from(winref, wi):
            base = pl.multiple_of(extract(pbase_v, wi), 8)
            cnt = extract(hcnt_v, wi)
            lo = wi * _WIN

            def chunk(j, _):
                p0 = pl.multiple_of(base + j * 16, 8)
                v16 = pv_v[pl.ds(p0, 16)]
                b16 = pb_v[pl.ds(p0, 16)]
                m = (v16 >> _SHIFT) == wi
                vloc = v16 - lo
                for dd in range(8):
                    dd16 = jnp.full((16,), dd, jnp.int32)
                    vals = plsc.load_gather(winref, [dd16, vloc], mask=m)
                    plsc.store_scatter(out_v, [dd16, b16], vals, mask=m)
                return 0

            lax.fori_loop(0, (cnt + 15) >> 4, chunk, 0)

        def run_item(k, item):
            f = item >> 2
            ds = item & 3

            @pl.when(k == 0)
            def _():
                fire_window(item, 0, k)

            xoff = pl.multiple_of(f * _B, 128)
            pltpu.sync_copy(x_hbm.at[pl.ds(xoff, _B)], xrow_v)
            xrow = xrow_v

            for c in range(4):
                hcnt_v[pl.ds(c * 16, 16)] = zeros16

            def hist(i, _):
                v16 = xrow[pl.ds(i * 16, 16)]
                bk = v16 >> _SHIFT
                cnt16, lm = plsc.scan_count(bk)
                plsc.addupdate_scatter(
                    hcnt_v, [bk], cnt16 - (_CNT_BASE - 1), mask=lm)
                return 0

            lax.fori_loop(0, _CHUNKS, hist, 0)

            carry = jnp.int32(0)
            for c in range(4):
                h16 = hcnt_v[pl.ds(c * 16, 16)]
                pc = (h16 + 15) & jnp.int32(-16)
                ex = plsc.cumsum(pc) - pc + carry
                pbase_v[pl.ds(c * 16, 16)] = ex
                hrun_v[pl.ds(c * 16, 16)] = ex
                carry = carry + jnp.sum(pc)

            def fill(i, _):
                pv_v[pl.ds(i * 16, 16)] = zeros16
                pb_v[pl.ds(i * 16, 16)] = zeros16 + _B
                return 0

            lax.fori_loop(0, _PAIR_CHUNKS, fill, 0)

            def scat(i, _):
                v16 = xrow[pl.ds(i * 16, 16)]
                b16 = i * 16 + iota
                bk = v16 >> _SHIFT
                cnt16, lm = plsc.scan_count(bk)
                pos = plsc.load_gather(hrun_v, [bk]) + (cnt16 - _CNT_BASE)
                plsc.store_scatter(pv_v, [pos], v16)
                plsc.store_scatter(pb_v, [pos], b16)
                plsc.addupdate_scatter(
                    hrun_v, [bk], cnt16 - (_CNT_BASE - 1), mask=lm)
                return 0

            lax.fori_loop(0, _CHUNKS, scat, 0)

            @pl.when(k > 0)
            def _():
                pltpu.make_async_copy(out_hbm.at[0, 0],
                                      out_v.at[:, pl.ds(0, _B)],
                                      s_out).wait()

            def wloop(wi, _):
                drain_window(True)
                fire_window(item, wi + 1, k)
                gather_window(wi, k)
                return 0

            lax.fori_loop(0, _NFULL, wloop, 0)
            drain_window(False)

            nxt = item + 32

            @pl.when(nxt < _N_ITEMS)
            def _():
                fire_window(nxt, 0, k + 1)

            gather_from(tail_v, _NFULL)
            pltpu.async_copy(out_v.at[:, pl.ds(0, _B)], out_hbm.at[f, ds],
                             s_out)

        def item_loop(k, _):
            item = k * 32 + w

            @pl.when(item < _N_ITEMS)
            def _():
                run_item(k, item)

            return 0

        lax.fori_loop(0, 4, item_loop, 0)
        pltpu.make_async_copy(out_hbm.at[0, 0], out_v.at[:, pl.ds(0, _B)],
                              s_out).wait()

    tabs = jnp.transpose(tables, (0, 2, 1)).reshape(_N_FIELDS, 4, 8, _VOCAB)
    out4 = gather_kernel(x.reshape(_N_FIELDS * _B), tabs)
    return jnp.transpose(out4.reshape(_N_FIELDS, _D, _B), (2, 0, 1))


# speedup vs baseline: 6.8234x; 1.7334x over previous
import functools

import jax
import jax.numpy as jnp
from jax import lax
from jax.experimental import pallas as pl
from jax.experimental.pallas import tpu as pltpu
from jax.experimental.pallas import tpu_sc as plsc

_N_FIELDS = 26
_VOCAB = 100000
_D = 32
_B = 4096
_WIN = 2048
_NFULL = _VOCAB // _WIN
_TAIL = _VOCAB - _NFULL * _WIN
_NWIN = _NFULL + 1
_SHIFT = 11
_NBUF = 4
_CHUNKS = _B // 16
_PAIR_CAP = _B + _NWIN * 15 + 9
_N_ITEMS = _N_FIELDS * 4
_CNT_BASE = 1


def kernel(x, tables):
    info = plsc.get_sparse_core_info()
    nc = info.num_cores

    mesh = plsc.VectorSubcoreMesh(core_axis_name="c", subcore_axis_name="s")

    @functools.partial(
        pl.kernel,
        mesh=mesh,
        out_type=jax.ShapeDtypeStruct((_N_FIELDS, 4, 8, _B), jnp.float32),
        compiler_params=pltpu.CompilerParams(needs_layout_passes=False),
        scratch_types=[
            pltpu.VMEM((_NBUF, 8, _WIN), jnp.float32),
            pltpu.VMEM((8, _TAIL), jnp.float32),
            pltpu.VMEM((8, _B), jnp.float32),
            pltpu.VMEM((_B,), jnp.int32),
            pltpu.VMEM((_PAIR_CAP,), jnp.int32),
            pltpu.VMEM((_PAIR_CAP,), jnp.int32),
            pltpu.VMEM((64,), jnp.int32),
            pltpu.VMEM((64,), jnp.int32),
            pltpu.VMEM((64,), jnp.int32),
            pltpu.SemaphoreType.DMA,
            pltpu.SemaphoreType.DMA,
        ],
    )
    def gather_kernel(x_hbm, tab_hbm, out_hbm, win_v, tail_v, out_v, xrow_v,
                      pv_v, pb_v, hcnt_v, pbase_v, hrun_v, s_win, s_out):
        w = lax.axis_index("s") * nc + lax.axis_index("c")
        iota = lax.iota(jnp.int32, 16)
        zeros16 = jnp.zeros((16,), jnp.int32)

        def extract(vec64_ref, j):
            c = pl.multiple_of((j >> 4) * 16, 8)
            v16 = vec64_ref[pl.ds(c, 16)]
            return jnp.max(jnp.where(iota == (j & 15), v16, 0))

        def fire_window(item, wi):
            f = item >> 2
            ds = item & 3

            @pl.when(wi < _NFULL)
            def _():
                off = pl.multiple_of(wi * _WIN, 128)
                pltpu.async_copy(
                    tab_hbm.at[f, ds, :, pl.ds(off, _WIN)],
                    win_v.at[wi & (_NBUF - 1)], s_win)

            @pl.when(wi == _NFULL)
            def _():
                pltpu.async_copy(
                    tab_hbm.at[f, ds, :, pl.ds(_NFULL * _WIN, _TAIL)],
                    tail_v, s_win)

        def drain_window(full):
            if full:
                pltpu.make_async_copy(
                    tab_hbm.at[0, 0, :, pl.ds(0, _WIN)], win_v.at[0],
                    s_win).wait()
            else:
                pltpu.make_async_copy(
                    tab_hbm.at[0, 0, :, pl.ds(_NFULL * _WIN, _TAIL)],
                    tail_v, s_win).wait()

        def gather_from(winref, wi):
            base = pl.multiple_of(extract(pbase_v, wi), 8)
            cnt = extract(hcnt_v, wi)
            lo = wi * _WIN

            def chunk(j, _):
                p0 = pl.multiple_of(base + j * 16, 8)
                v16 = pv_v[pl.ds(p0, 16)]
                b16 = pb_v[pl.ds(p0, 16)]
                m = (j * 16 + iota) < cnt
                vloc = v16 - lo
                for dd in range(8):
                    dd16 = jnp.full((16,), dd, jnp.int32)
                    vals = plsc.load_gather(winref, [dd16, vloc], mask=m)
                    plsc.store_scatter(out_v, [dd16, b16], vals, mask=m)
                return 0

            lax.fori_loop(0, (cnt + 15) >> 4, chunk, 0)

        def run_item(k, item):
            f = item >> 2
            ds = item & 3

            @pl.when(k == 0)
            def _():
                for wi in range(_NBUF):
                    fire_window(item, wi)

            xoff = pl.multiple_of(f * _B, 128)
            pltpu.sync_copy(x_hbm.at[pl.ds(xoff, _B)], xrow_v)

            for c in range(4):
                hcnt_v[pl.ds(c * 16, 16)] = zeros16

            def hist(i, _):
                v16 = xrow_v[pl.ds(i * 16, 16)]
                bk = v16 >> _SHIFT
                cnt16, lm = plsc.scan_count(bk)
                plsc.addupdate_scatter(
                    hcnt_v, [bk], cnt16 - (_CNT_BASE - 1), mask=lm)
                return 0

            lax.fori_loop(0, _CHUNKS, hist, 0)

            carry = jnp.int32(0)
            for c in range(4):
                h16 = hcnt_v[pl.ds(c * 16, 16)]
                pc = (h16 + 15) & jnp.int32(-16)
                ex = plsc.cumsum(pc) - pc + carry
                pbase_v[pl.ds(c * 16, 16)] = ex
                hrun_v[pl.ds(c * 16, 16)] = ex
                carry = carry + jnp.sum(pc)

            def scat(i, _):
                v16 = xrow_v[pl.ds(i * 16, 16)]
                b16 = i * 16 + iota
                bk = v16 >> _SHIFT
                cnt16, lm = plsc.scan_count(bk)
                pos = plsc.load_gather(hrun_v, [bk]) + (cnt16 - _CNT_BASE)
                plsc.store_scatter(pv_v, [pos], v16)
                plsc.store_scatter(pb_v, [pos], b16)
                plsc.addupdate_scatter(
                    hrun_v, [bk], cnt16 - (_CNT_BASE - 1), mask=lm)
                return 0

            lax.fori_loop(0, _CHUNKS, scat, 0)

            @pl.when(k > 0)
            def _():
                pltpu.make_async_copy(out_hbm.at[0, 0], out_v, s_out).wait()

            def wloop(wi, _):
                drain_window(True)
                gather_from(win_v.at[wi & (_NBUF - 1)], wi)
                fire_window(item, wi + _NBUF)
                return 0

            lax.fori_loop(0, _NFULL, wloop, 0)
            drain_window(False)

            nxt = item + 32

            @pl.when(nxt < _N_ITEMS)
            def _():
                for wi in range(_NBUF):
                    fire_window(nxt, wi)

            gather_from(tail_v, _NFULL)
            pltpu.async_copy(out_v, out_hbm.at[f, ds], s_out)

        def item_loop(k, _):
            item = k * 32 + w

            @pl.when(item < _N_ITEMS)
            def _():
                run_item(k, item)

            return 0

        lax.fori_loop(0, 4, item_loop, 0)
        pltpu.make_async_copy(out_hbm.at[0, 0], out_v, s_out).wait()

    tabs = jnp.transpose(tables, (0, 2, 1)).reshape(_N_FIELDS, 4, 8, _VOCAB)
    out4 = gather_kernel(x.reshape(_N_FIELDS * _B), tabs)
    return jnp.transpose(out4.reshape(_N_FIELDS, _D, _B), (2, 0, 1))


# speedup vs baseline: 6.8253x; 1.0003x over previous
import functools

import jax
import jax.numpy as jnp
from jax import lax
from jax.experimental import pallas as pl
from jax.experimental.pallas import tpu as pltpu
from jax.experimental.pallas import tpu_sc as plsc

_N_FIELDS = 26
_VOCAB = 100000
_D = 32
_B = 4096
_WIN = 1024
_NFULL = _VOCAB // _WIN
_TAIL = _VOCAB - _NFULL * _WIN
_NWIN = _NFULL + 1
_SHIFT = 10
_NBUF = 8
_CNT16 = (_NWIN + 15) // 16
_CHUNKS = _B // 16
_PAIR_CAP = _B + _NWIN * 15 + 2
_N_ITEMS = _N_FIELDS * 4
_CNT_BASE = 1


def kernel(x, tables):
    info = plsc.get_sparse_core_info()
    nc = info.num_cores

    mesh = plsc.VectorSubcoreMesh(core_axis_name="c", subcore_axis_name="s")

    @functools.partial(
        pl.kernel,
        mesh=mesh,
        out_type=jax.ShapeDtypeStruct((_N_FIELDS, 4, 8, _B), jnp.float32),
        compiler_params=pltpu.CompilerParams(needs_layout_passes=False),
        scratch_types=[
            pltpu.VMEM((_NBUF, 8, _WIN), jnp.float32),
            pltpu.VMEM((8, _TAIL), jnp.float32),
            pltpu.VMEM((8, _B), jnp.float32),
            pltpu.VMEM((_B,), jnp.int32),
            pltpu.VMEM((_PAIR_CAP,), jnp.int32),
            pltpu.VMEM((_PAIR_CAP,), jnp.int32),
            pltpu.VMEM((16 * _CNT16,), jnp.int32),
            pltpu.VMEM((16 * _CNT16,), jnp.int32),
            pltpu.VMEM((16 * _CNT16,), jnp.int32),
            pltpu.SemaphoreType.DMA,
            pltpu.SemaphoreType.DMA,
        ],
    )
    def gather_kernel(x_hbm, tab_hbm, out_hbm, win_v, tail_v, out_v, xrow_v,
                      pv_v, pb_v, hcnt_v, pbase_v, hrun_v, s_win, s_out):
        w = lax.axis_index("s") * nc + lax.axis_index("c")
        iota = lax.iota(jnp.int32, 16)
        zeros16 = jnp.zeros((16,), jnp.int32)

        def extract(vec64_ref, j):
            c = pl.multiple_of((j >> 4) * 16, 8)
            v16 = vec64_ref[pl.ds(c, 16)]
            return jnp.max(jnp.where(iota == (j & 15), v16, 0))

        def fire_window(item, wi):
            f = item >> 2
            ds = item & 3

            @pl.when(wi < _NFULL)
            def _():
                off = pl.multiple_of(wi * _WIN, 128)
                pltpu.async_copy(
                    tab_hbm.at[f, ds, :, pl.ds(off, _WIN)],
                    win_v.at[wi & (_NBUF - 1)], s_win)

            @pl.when(wi == _NFULL)
            def _():
                pltpu.async_copy(
                    tab_hbm.at[f, ds, :, pl.ds(_NFULL * _WIN, _TAIL)],
                    tail_v, s_win)

        def drain_window(full):
            if full:
                pltpu.make_async_copy(
                    tab_hbm.at[0, 0, :, pl.ds(0, _WIN)], win_v.at[0],
                    s_win).wait()
            else:
                pltpu.make_async_copy(
                    tab_hbm.at[0, 0, :, pl.ds(_NFULL * _WIN, _TAIL)],
                    tail_v, s_win).wait()

        def gather_from(winref, wi):
            base = pl.multiple_of(extract(pbase_v, wi), 8)
            cnt = extract(hcnt_v, wi)
            lo = wi * _WIN

            def chunk(j, _):
                p0 = pl.multiple_of(base + j * 16, 8)
                v16 = pv_v[pl.ds(p0, 16)]
                b16 = pb_v[pl.ds(p0, 16)]
                m = (j * 16 + iota) < cnt
                vloc = v16 - lo
                for dd in range(8):
                    dd16 = jnp.full((16,), dd, jnp.int32)
                    vals = plsc.load_gather(winref, [dd16, vloc], mask=m)
                    plsc.store_scatter(out_v, [dd16, b16], vals, mask=m)
                return 0

            lax.fori_loop(0, (cnt + 15) >> 4, chunk, 0)

        def run_item(k, item):
            f = item >> 2
            ds = item & 3

            @pl.when(k == 0)
            def _():
                for wi in range(_NBUF):
                    fire_window(item, wi)

            xoff = pl.multiple_of(f * _B, 128)
            pltpu.sync_copy(x_hbm.at[pl.ds(xoff, _B)], xrow_v)

            for c in range(_CNT16):
                hcnt_v[pl.ds(c * 16, 16)] = zeros16

            def hist(i, _):
                v16 = xrow_v[pl.ds(i * 16, 16)]
                bk = v16 >> _SHIFT
                cnt16, lm = plsc.scan_count(bk)
                plsc.addupdate_scatter(
                    hcnt_v, [bk], cnt16 - (_CNT_BASE - 1), mask=lm)
                return 0

            lax.fori_loop(0, _CHUNKS, hist, 0)

            carry = jnp.int32(0)
            for c in range(_CNT16):
                h16 = hcnt_v[pl.ds(c * 16, 16)]
                pc = (h16 + 15) & jnp.int32(-16)
                ex = plsc.cumsum(pc) - pc + carry
                pbase_v[pl.ds(c * 16, 16)] = ex
                hrun_v[pl.ds(c * 16, 16)] = ex
                carry = carry + jnp.sum(pc)

            def scat(i, _):
                v16 = xrow_v[pl.ds(i * 16, 16)]
                b16 = i * 16 + iota
                bk = v16 >> _SHIFT
                cnt16, lm = plsc.scan_count(bk)
                pos = plsc.load_gather(hrun_v, [bk]) + (cnt16 - _CNT_BASE)
                plsc.store_scatter(pv_v, [pos], v16)
                plsc.store_scatter(pb_v, [pos], b16)
                plsc.addupdate_scatter(
                    hrun_v, [bk], cnt16 - (_CNT_BASE - 1), mask=lm)
                return 0

            lax.fori_loop(0, _CHUNKS, scat, 0)

            @pl.when(k > 0)
            def _():
                pltpu.make_async_copy(out_hbm.at[0, 0], out_v, s_out).wait()

            def wloop(wi, _):
                drain_window(True)
                gather_from(win_v.at[wi & (_NBUF - 1)], wi)
                fire_window(item, wi + _NBUF)
                return 0

            lax.fori_loop(0, _NFULL, wloop, 0)
            drain_window(False)

            nxt = item + 32

            @pl.when(nxt < _N_ITEMS)
            def _():
                for wi in range(_NBUF):
                    fire_window(nxt, wi)

            gather_from(tail_v, _NFULL)
            pltpu.async_copy(out_v, out_hbm.at[f, ds], s_out)

        def item_loop(k, _):
            item = k * 32 + w

            @pl.when(item < _N_ITEMS)
            def _():
                run_item(k, item)

            return 0

        lax.fori_loop(0, 4, item_loop, 0)
        pltpu.make_async_copy(out_hbm.at[0, 0], out_v, s_out).wait()

    tabs = jnp.transpose(tables, (0, 2, 1)).reshape(_N_FIELDS, 4, 8, _VOCAB)
    out4 = gather_kernel(x.reshape(_N_FIELDS * _B), tabs)
    return jnp.transpose(out4.reshape(_N_FIELDS, _D, _B), (2, 0, 1))
